# Initial kernel scaffold; baseline (speedup 1.0000x reference)
import jax
import jax.numpy as jnp
from jax.experimental import pallas as pl


def kernel(features, labels):
    raise NotImplementedError("write your pallas kernel here")


# speedup vs baseline: 4.7412x; 4.7412x over previous
import functools

import numpy as np
import jax
import jax.numpy as jnp
from jax import lax
from jax.experimental import pallas as pl
from jax.experimental.pallas import tpu as pltpu
import jax.experimental.pallas.tpu_sc as plsc

_T = 0.1
_P = 2048
_NMAX = 10240
_K = 5120
_S = _P + _NMAX
_EPS = 1e-8
_NEG = -1e9

_NSUB = 16
_SPAD = _S + 8
_JT = 1024
_R = 64


def _code_body(lab_ref, code_ref):
    x = lab_ref[0]
    fg = x > 0
    f = fg.astype(jnp.float32)
    H, W = f.shape
    zc = jnp.zeros((H, 5), jnp.float32)
    p = jnp.concatenate([zc, f, zc], axis=1)
    m = f
    for k in range(11):
        m = jnp.maximum(m, lax.slice(p, (0, k), (H, k + W)))
    zr = jnp.zeros((5, W), jnp.float32)
    p2 = jnp.concatenate([zr, m, zr], axis=0)
    d = m
    for k in range(11):
        d = jnp.maximum(d, lax.slice(p2, (k, 0), (k + H, W)))
    rim = (d > 0.0) & (~fg)
    code_ref[0] = jnp.where(fg, 1, jnp.where(rim, 2, 0)).astype(jnp.int32)


def _label_code(labels_i32):
    B, H, W = labels_i32.shape
    return pl.pallas_call(
        _code_body,
        grid=(B,),
        in_specs=[pl.BlockSpec((1, H, W), lambda b: (b, 0, 0))],
        out_specs=pl.BlockSpec((1, H, W), lambda b: (b, 0, 0)),
        out_shape=jax.ShapeDtypeStruct((B, H, W), jnp.int32),
    )(labels_i32)


def _compact_body(code_hbm, sel_hbm, cnt_hbm,
                  codebuf, fgbuf, rimbuf, cntbuf, allcnt, zbuf, idxbuf,
                  cntout, shcnt):
    b = lax.axis_index("c")
    sid = lax.axis_index("s")
    HW = 512 * 512
    chunk = HW // _NSUB
    iota = lax.iota(jnp.int32, 16)

    pltpu.sync_copy(code_hbm.at[pl.ds(b * HW + sid * chunk, chunk)], codebuf)

    def body(i, carry):
        c1, c2 = carry
        v = codebuf[pl.ds(i * 16, 16)]
        pix = sid * chunk + i * 16 + iota
        m1 = v == 1
        i1 = m1.astype(jnp.int32)
        cs1 = plsc.cumsum(i1)
        plsc.store_scatter(fgbuf, [c1 + cs1 - i1], pix, mask=m1)
        m2 = v == 2
        i2 = m2.astype(jnp.int32)
        cs2 = plsc.cumsum(i2)
        plsc.store_scatter(rimbuf, [c2 + cs2 - i2], pix, mask=m2)
        return c1 + cs1[15], c2 + cs2[15]

    c1, c2 = lax.fori_loop(0, chunk // 16, body, (jnp.int32(0), jnp.int32(0)))

    zbuf[...] = jnp.zeros((16,), jnp.int32)
    zshare = _S // _NSUB
    for g in range(zshare // 16):
        pltpu.sync_copy(
            zbuf, sel_hbm.at[pl.ds(b * _SPAD + sid * zshare + g * 16, 16)])

    cntbuf[...] = jnp.where(iota == 0, c1, jnp.where(iota == 1, c2, 0))
    pltpu.sync_copy(cntbuf, shcnt.at[sid])
    plsc.subcore_barrier()
    pltpu.sync_copy(shcnt, allcnt)
    pfx1 = jnp.int32(0)
    pfx2 = jnp.int32(0)
    tot1 = jnp.int32(0)
    tot2 = jnp.int32(0)
    for r in range(_NSUB):
        row = allcnt[r]
        a1 = row[0]
        a2 = row[1]
        mine = jnp.int32(r) < sid
        pfx1 = pfx1 + jnp.where(mine, a1, 0)
        pfx2 = pfx2 + jnp.where(mine, a2, 0)
        tot1 = tot1 + a1
        tot2 = tot2 + a2

    @pl.when(sid == 0)
    def _():
        cntout[...] = jnp.where(iota == 0, tot1, jnp.where(iota == 1, tot2, 0))
        pltpu.sync_copy(cntout, cnt_hbm.at[pl.ds(b * 16, 16)])

    def scatter(buf, pfx, cnt, limit, out_base):
        n = jnp.minimum(jnp.maximum(limit - pfx, 0), cnt)

        dummy = b * _SPAD + _S

        def chunk_body(j, _):
            @pl.when(j * 128 < n)
            def _():
                for g in range(8):
                    slot = j * 128 + g * 16 + iota
                    tgt = jnp.where(slot < n, out_base + pfx + slot, dummy)
                    idxbuf[pl.ds(g * 16, 16)] = tgt
                pltpu.sync_copy(
                    buf.at[pl.ds(j * 128, 128)],
                    sel_hbm.at[idxbuf])
            return 0

        lax.fori_loop(0, chunk // 128, chunk_body, 0)

    scatter(fgbuf, pfx1, c1, jnp.int32(_P), b * _SPAD)
    scatter(rimbuf, pfx2, c2, jnp.int32(_NMAX), b * _SPAD + _P)


def _compact(code_flat, B):
    HW = 512 * 512
    chunk = HW // _NSUB
    mesh = plsc.VectorSubcoreMesh(core_axis_name="c", subcore_axis_name="s")
    return pl.kernel(
        _compact_body,
        out_type=(
            jax.ShapeDtypeStruct((B * _SPAD,), jnp.int32),
            jax.ShapeDtypeStruct((B * 16,), jnp.int32),
        ),
        mesh=mesh,
        compiler_params=pltpu.CompilerParams(needs_layout_passes=False),
        scratch_types=[
            pltpu.VMEM((chunk,), jnp.int32),
            pltpu.VMEM((chunk + 16,), jnp.int32),
            pltpu.VMEM((chunk + 16,), jnp.int32),
            pltpu.VMEM((16,), jnp.int32),
            pltpu.VMEM((_NSUB, 16), jnp.int32),
            pltpu.VMEM((16,), jnp.int32),
            pltpu.VMEM((128,), jnp.int32),
            pltpu.VMEM((16,), jnp.int32),
            pltpu.VMEM_SHARED((_NSUB, 16), jnp.int32),
        ],
    )(code_flat)


def _gather_body(feat_hbm, sel_hbm, g_hbm,
                 selbuf, rowsb, laneb, urank, urows, rowbuf, outc, idxc, sem):
    b = lax.axis_index("c")
    sid = lax.axis_index("s")
    C = 96
    cpt = C // _NSUB
    iota = lax.iota(jnp.int32, 16)

    pltpu.sync_copy(sel_hbm.at[pl.ds(b * _SPAD, _S)], selbuf)

    def meta(k, u):
        sl = selbuf[pl.ds(k * 16, 16)]
        rows = lax.shift_right_logical(sl, 9)
        laneb[pl.ds(k * 16, 16)] = jnp.bitwise_and(sl, 511)
        rowsb[pl.ds(k * 16, 16)] = rows
        pidx = k * 16 - 1 + iota
        mok = pidx >= 0
        prev = plsc.load_gather(rowsb, [jnp.maximum(pidx, 0)], mask=mok)
        new = (rows != prev) | (~mok)
        ni = new.astype(jnp.int32)
        cs = plsc.cumsum(ni)
        rank = u + cs - ni
        urank[pl.ds(k * 16, 16)] = rank
        plsc.store_scatter(urows, [rank], rows, mask=new)
        return u + cs[15]

    U = lax.fori_loop(0, _S // 16, meta, jnp.int32(0))

    def chan(ci, _):
        c = sid * cpt + ci
        base_row = (b * C + c) * 512

        def chunk_body(q, _):
            @pl.when(q * _R < U)
            def _():
                for g in range(_R // 16):
                    slot = q * _R + g * 16 + iota
                    uv = urows[pl.ds(q * _R + g * 16, 16)]
                    idxc[pl.ds(g * 16, 16)] = base_row + jnp.where(
                        slot < U, uv, 0)
                pltpu.async_copy(
                    feat_hbm.at[idxc], rowbuf, sem).wait()

                def extract(k, _):
                    ur = urank[pl.ds(k * 16, 16)]
                    lr = ur - q * _R
                    m = (lr >= 0) & (lr < _R)
                    lrc = jnp.minimum(jnp.maximum(lr, 0), _R - 1)
                    ln = laneb[pl.ds(k * 16, 16)]
                    vals = plsc.load_gather(rowbuf, [lrc, ln], mask=m)
                    cur = outc[pl.ds(k * 16, 16)]
                    outc[pl.ds(k * 16, 16)] = jnp.where(m, vals, cur)
                    return 0

                lax.fori_loop(0, _S // 16, extract, 0)
            return 0

        lax.fori_loop(0, _S // _R, chunk_body, 0)
        pltpu.sync_copy(outc, g_hbm.at[pl.ds((b * C + c) * _S, _S)])
        return 0

    lax.fori_loop(0, cpt, chan, 0)


def _gather(feat2d, sel, B, C):
    mesh = plsc.VectorSubcoreMesh(core_axis_name="c", subcore_axis_name="s")
    return pl.kernel(
        _gather_body,
        out_type=jax.ShapeDtypeStruct((B * C * _S,), jnp.float32),
        mesh=mesh,
        compiler_params=pltpu.CompilerParams(needs_layout_passes=False),
        scratch_types=[
            pltpu.VMEM((_S,), jnp.int32),
            pltpu.VMEM((_S,), jnp.int32),
            pltpu.VMEM((_S,), jnp.int32),
            pltpu.VMEM((_S,), jnp.int32),
            pltpu.VMEM((_S + 16,), jnp.int32),
            pltpu.VMEM((_R, 512), jnp.float32),
            pltpu.VMEM((_S,), jnp.float32),
            pltpu.VMEM((_R,), jnp.int32),
            pltpu.SemaphoreType.DMA,
        ],
    )(feat2d, sel)


def _monokey(x):
    u = lax.bitcast_convert_type(x, jnp.int32)
    return jnp.where(
        u >= 0, u,
        jnp.bitwise_xor(jnp.bitwise_not(u), jnp.int32(-2147483648)))


_BITS = [np.int32(np.uint32(1 << b)) for b in range(31, -1, -1)]


def _select_topk(scores2d, k_i32):
    key = _monokey(scores2d)
    ts = jnp.int32(-2147483648)
    for inc in _BITS:
        cand = ts + jnp.int32(inc)
        c = jnp.sum((key >= cand).astype(jnp.int32))
        ts = jnp.where(c >= k_i32, cand, ts)
    gt = key > ts
    eq = key == ts
    slots = (k_i32 - jnp.sum(gt.astype(jnp.int32))).astype(jnp.float32)
    E = eq.astype(jnp.float32)
    r, l = E.shape
    Ut = (lax.broadcasted_iota(jnp.int32, (l, l), 0)
          <= lax.broadcasted_iota(jnp.int32, (l, l), 1)).astype(jnp.float32)
    incl = jax.lax.dot_general(E, Ut, (((1,), (0,)), ((), ())),
                               preferred_element_type=jnp.float32)
    Lt = (lax.broadcasted_iota(jnp.int32, (r, r), 1)
          < lax.broadcasted_iota(jnp.int32, (r, r), 0)).astype(jnp.float32)
    rowsum = jnp.sum(E, axis=1, keepdims=True)
    off = jax.lax.dot_general(Lt, rowsum, (((1,), (0,)), ((), ())),
                              preferred_element_type=jnp.float32)
    tierank = incl + off - E
    return gt | (eq & (tierank < slots))


def _dotT(a, b):
    return jax.lax.dot_general(a, b, (((0,), (0,)), ((), ())),
                               preferred_element_type=jnp.float32)


def _loss_body(g_ref, cnt_ref, out_ref, gn_ref, w_ref):
    v_p = jnp.minimum(cnt_ref[0, 0, 0], _P)
    v_c = jnp.minimum(cnt_ref[0, 0, 1], _NMAX)

    G = g_ref[0]
    inv = 1.0 / (jnp.sqrt(jnp.sum(G * G, axis=0, keepdims=True)) + _EPS)
    gn_ref[...] = G * inv
    Gp = gn_ref[:, :_P]

    irow = lax.broadcasted_iota(jnp.int32, (_P, _JT), 0)

    def cm_tile(jt, _):
        off = pl.multiple_of(_P + jt * _JT, _JT)
        s = _dotT(Gp, gn_ref[:, pl.ds(off, _JT)])
        s = jnp.where(irow < v_p, s, _NEG)
        woff = pl.multiple_of(jt * _JT, _JT)
        w_ref[0, pl.ds(woff, _JT)] = jnp.max(s, axis=0)
        return 0

    lax.fori_loop(0, _NMAX // _JT, cm_tile, 0, unroll=False)
    colmax = w_ref[0, :_NMAX].reshape(80, 128)

    jc = (lax.broadcasted_iota(jnp.int32, (80, 128), 0) * 128
          + lax.broadcasted_iota(jnp.int32, (80, 128), 1))
    valid = jc < v_c
    hard = jnp.where(valid, colmax, _NEG)
    easy = jnp.where(valid, -colmax, _NEG)
    w = (_select_topk(hard, jnp.int32(_K)).astype(jnp.float32)
         + _select_topk(easy, jnp.int32(_K)).astype(jnp.float32))
    w_ref[0, :_NMAX] = jnp.where(valid, w, 0.0).reshape(1, _NMAX)[0]
    w_ref[0, _NMAX:] = jnp.zeros((_S - _NMAX,), jnp.float32)

    def dn_tile(jt, carry):
        D, N = carry
        off = pl.multiple_of(jt * _JT, _JT)
        s = _dotT(Gp, gn_ref[:, pl.ds(off, _JT)]) * (1.0 / _T)
        jglob = lax.broadcasted_iota(jnp.int32, (_P, _JT), 1) + jt * _JT
        m = (jglob < v_p) & (jglob != irow)
        woff = pl.multiple_of(
            jnp.where(jt >= _P // _JT, jt * _JT - _P, _NMAX), _JT)
        wt = w_ref[0, pl.ds(woff, _JT)].reshape(1, _JT)
        mf = m.astype(jnp.float32)
        e = jnp.exp(s)
        D = D + jnp.sum(e * (mf + wt), axis=1, keepdims=True)
        N = N + jnp.sum(jnp.where(m, s, 0.0), axis=1, keepdims=True)
        return D, N

    D, N = lax.fori_loop(
        0, _S // _JT, dn_tile,
        (jnp.zeros((_P, 1), jnp.float32), jnp.zeros((_P, 1), jnp.float32)),
        unroll=False)

    v_pf = v_p.astype(jnp.float32)
    ir = lax.broadcasted_iota(jnp.int32, (_P, 1), 0)
    contrib = (v_pf - 1.0) * jnp.log(D) - N
    loss_sum = jnp.sum(jnp.where(ir < v_p, contrib, 0.0))
    npair = jnp.maximum(v_pf * v_pf - v_pf, 1.0)
    out_ref[...] = jnp.where(v_p > 1, loss_sum / npair, 0.0).reshape(1, 1, 1)


def _loss(g3d, counts, B, C):
    return pl.pallas_call(
        _loss_body,
        grid=(B,),
        in_specs=[
            pl.BlockSpec((1, C, _S), lambda b: (b, 0, 0)),
            pl.BlockSpec((1, 1, 16), lambda b: (b, 0, 0),
                         memory_space=pltpu.SMEM),
        ],
        out_specs=pl.BlockSpec((1, 1, 1), lambda b: (b, 0, 0)),
        out_shape=jax.ShapeDtypeStruct((B, 1, 1), jnp.float32),
        scratch_shapes=[
            pltpu.VMEM((C, _S), jnp.float32),
            pltpu.VMEM((1, _S), jnp.float32),
        ],
    )(g3d, counts)


def kernel(features, labels):
    B, C, H, W = features.shape
    code = _label_code(labels.astype(jnp.int32))
    code_flat = code.reshape(B * H * W)
    sel, counts = _compact(code_flat, B)
    feat2d = features.reshape(B * C * H, W)
    g_flat = _gather(feat2d, sel, B, C)
    g3d = g_flat.reshape(B, C, _S)
    losses = _loss(g3d, counts.reshape(B, 1, 16), B, C)
    return jnp.mean(losses[:, 0, 0])
